# baseline (speedup 1.0000x reference)
import functools

import jax
import jax.numpy as jnp
from jax import lax
from jax.experimental import pallas as pl
from jax.experimental.pallas import tpu as pltpu
from jax.experimental.pallas import tpu_sc as plsc

_K = 10
_NKP = 9
_C = 1024
_BIG = 1e30


def _sc_scan_body(n_points, kpts_hbm, cpt_hbm, pcld_hbm, seg_hbm, out_hbm,
                  kc, cc, pc, sgc, st, bk_ref, th_ref, sem):
    b = lax.axis_index("s") * 2 + lax.axis_index("c")
    it = lax.iota(jnp.int32, 16)
    big = jnp.float32(_BIG)

    for kk in range(_NKP):
        bk_ref[pl.ds(kk * 16, 16)] = jnp.full((16,), big, jnp.float32)
        th_ref[kk] = big

    def chunk_body(q, carry):
        s = q * _C
        cps = (pltpu.async_copy(kpts_hbm.at[b, :, :, pl.ds(s, _C)], kc, sem),
               pltpu.async_copy(cpt_hbm.at[b, :, pl.ds(s, _C)], cc, sem),
               pltpu.async_copy(pcld_hbm.at[:, b, pl.ds(s, _C)], pc, sem),
               pltpu.async_copy(seg_hbm.at[b, :, pl.ds(s, _C)], sgc, sem))
        for cp in cps:
            cp.wait()

        def group_body(j, carry):
            sl = pl.ds(j * 16, 16)
            gp = q * _C + j * 16 + it
            s0 = sgc[0, sl]
            s1 = sgc[1, sl]
            obj = s1 > s0
            sent = jnp.float32(1e18) + gp.astype(jnp.float32) * jnp.float32(1e12)
            for kk in range(_NKP):
                if kk < 8:
                    x = kc[0, kk, sl]
                    y = kc[1, kk, sl]
                    z = kc[2, kk, sl]
                else:
                    x = cc[0, sl]
                    y = cc[1, sl]
                    z = cc[2, sl]
                n2 = x * x + y * y + z * z
                key = jnp.where(obj, n2, sent)
                m = jnp.min(key)

                @pl.when(m < th_ref[kk])
                def _(kk=kk, key=key, x=x, y=y, z=z, sl=sl):
                    cx = pc[0, sl] + x
                    cy = pc[1, sl] + y
                    cz = pc[2, sl] + z
                    bk = bk_ref[pl.ds(kk * 16, 16)]
                    bx = st[pl.ds(kk * 16, 16)]
                    by = st[pl.ds((_NKP + kk) * 16, 16)]
                    bz = st[pl.ds((2 * _NKP + kk) * 16, 16)]
                    sk, sx = plsc.sort_key_val(key, cx)
                    _, sy = plsc.sort_key_val(key, cy)
                    _, sz = plsc.sort_key_val(key, cz)
                    rk = lax.rev(sk, (0,))
                    take = bk <= rk
                    lo_k = jnp.where(take, bk, rk)
                    lo_x = jnp.where(take, bx, lax.rev(sx, (0,)))
                    lo_y = jnp.where(take, by, lax.rev(sy, (0,)))
                    lo_z = jnp.where(take, bz, lax.rev(sz, (0,)))
                    nk, nx = plsc.sort_key_val(lo_k, lo_x)
                    _, ny = plsc.sort_key_val(lo_k, lo_y)
                    _, nz = plsc.sort_key_val(lo_k, lo_z)
                    bk_ref[pl.ds(kk * 16, 16)] = nk
                    st[pl.ds(kk * 16, 16)] = nx
                    st[pl.ds((_NKP + kk) * 16, 16)] = ny
                    st[pl.ds((2 * _NKP + kk) * 16, 16)] = nz
                    th_ref[kk] = jnp.min(jnp.where(it == 9, nk, big))
            return 0

        return lax.fori_loop(0, _C // 16, group_body, 0)

    lax.fori_loop(0, n_points // _C, chunk_body, 0)
    pltpu.sync_copy(st, out_hbm.at[b])


def _cluster_kernel(g_ref, o_ref):
    g = g_ref[...]
    third = g.shape[0] // 3
    x = g[:third]
    y = g[third:2 * third]
    z = g[2 * third:]
    valid = lax.broadcasted_iota(jnp.int32, x.shape, 1) < _K
    inv_k = jnp.float32(1.0 / _K)
    eps = jnp.float32(1e-9)

    def stats(v):
        mean = jnp.sum(jnp.where(valid, v, 0.0), axis=1, keepdims=True) * inv_k
        d = v - mean
        std = jnp.sqrt(
            jnp.sum(jnp.where(valid, d * d, 0.0), axis=1, keepdims=True) * inv_k)
        return d, std

    dx, sx = stats(x)
    dy, sy = stats(y)
    dz, sz = stats(z)
    inl = (valid & (jnp.abs(dx) <= sx + eps) & (jnp.abs(dy) <= sy + eps)
           & (jnp.abs(dz) <= sz + eps))
    w = inl.astype(jnp.float32)
    denom = jnp.sum(w, axis=1, keepdims=True) + jnp.float32(1e-8)
    ox = jnp.sum(jnp.where(valid, x, 0.0) * w, axis=1, keepdims=True) / denom
    oy = jnp.sum(jnp.where(valid, y, 0.0) * w, axis=1, keepdims=True) / denom
    oz = jnp.sum(jnp.where(valid, z, 0.0) * w, axis=1, keepdims=True) / denom
    o_ref[...] = jnp.concatenate([ox, oy, oz], axis=1)


def kernel(pcld_input, kpts_pre_input, cpt_pre_input, seg_pre_input):
    b, n = pcld_input.shape[0], pcld_input.shape[1]
    kpts_t = jnp.transpose(kpts_pre_input, (0, 3, 2, 1))
    cpt_t = jnp.transpose(cpt_pre_input, (0, 3, 2, 1)).reshape(b, 3, n)
    pcld_t = jnp.transpose(pcld_input, (2, 0, 1))
    seg_t = jnp.transpose(seg_pre_input, (0, 2, 1))

    mesh = plsc.VectorSubcoreMesh(core_axis_name="c", subcore_axis_name="s")
    scan = pl.kernel(
        functools.partial(_sc_scan_body, n),
        mesh=mesh,
        compiler_params=pltpu.CompilerParams(needs_layout_passes=False),
        out_type=jax.ShapeDtypeStruct((b, 3 * _NKP * 16), jnp.float32),
        scratch_types=[
            pltpu.VMEM((3, 8, _C), jnp.float32),
            pltpu.VMEM((3, _C), jnp.float32),
            pltpu.VMEM((3, _C), jnp.float32),
            pltpu.VMEM((2, _C), jnp.float32),
            pltpu.VMEM((3 * _NKP * 16,), jnp.float32),
            pltpu.VMEM((_NKP * 16,), jnp.float32),
            pltpu.SMEM((16,), jnp.float32),
            pltpu.SemaphoreType.DMA,
        ],
    )
    g = scan(kpts_t, cpt_t, pcld_t, seg_t)
    g2 = g.reshape(b, 3, _NKP, 16).transpose(1, 0, 2, 3).reshape(3 * b * _NKP, 16)

    out = pl.pallas_call(
        _cluster_kernel,
        out_shape=jax.ShapeDtypeStruct((b * _NKP, 3), jnp.float32),
    )(g2)
    return out.reshape(b, _NKP, 3)


# speedup vs baseline: 3.4856x; 3.4856x over previous
import functools

import jax
import jax.numpy as jnp
from jax import lax
from jax.experimental import pallas as pl
from jax.experimental.pallas import tpu as pltpu
from jax.experimental.pallas import tpu_sc as plsc

_K = 10
_NKP = 9
_C = 1024
_BIG = 1e30


def _sc_scan_body(n_points, kpts_hbm, cpt_hbm, pcld_hbm, seg_hbm, out_hbm,
                  kc, cc, pc, sgc, st, bk_ref, sem):
    b = lax.axis_index("s") * 2 + lax.axis_index("c")
    it = lax.iota(jnp.int32, 16)
    big = jnp.float32(_BIG)

    for kk in range(_NKP):
        bk_ref[pl.ds(kk * 16, 16)] = jnp.full((16,), big, jnp.float32)

    def chunk_body(q, carry):
        s = q * _C
        cps = (pltpu.async_copy(kpts_hbm.at[b, :, :, pl.ds(s, _C)], kc, sem),
               pltpu.async_copy(cpt_hbm.at[b, :, pl.ds(s, _C)], cc, sem),
               pltpu.async_copy(pcld_hbm.at[:, b, pl.ds(s, _C)], pc, sem),
               pltpu.async_copy(seg_hbm.at[b, :, pl.ds(s, _C)], sgc, sem))
        for cp in cps:
            cp.wait()

        def group_body(j, carry):
            sl = pl.ds(j * 16, 16)
            gp = q * _C + j * 16 + it
            s0 = sgc[0, sl]
            s1 = sgc[1, sl]
            obj = s1 > s0
            sent = jnp.float32(1e18) + gp.astype(jnp.float32) * jnp.float32(1e12)
            keys = []
            hit = None
            for kk in range(_NKP):
                if kk < 8:
                    x = kc[0, kk, sl]
                    y = kc[1, kk, sl]
                    z = kc[2, kk, sl]
                else:
                    x = cc[0, sl]
                    y = cc[1, sl]
                    z = cc[2, sl]
                n2 = x * x + y * y + z * z
                key = jnp.where(obj, n2, sent)
                keys.append((key, x, y, z))
                thv = plsc.load_gather(
                    bk_ref, [jnp.full((16,), kk * 16 + 9, jnp.int32)])
                h = key < thv
                hit = h if hit is None else (hit | h)

            @pl.when(jnp.any(hit))
            def _(keys=keys, sl=sl):
                p0 = pc[0, sl]
                p1 = pc[1, sl]
                p2 = pc[2, sl]
                for kk in range(_NKP):
                    key, x, y, z = keys[kk]
                    cx = p0 + x
                    cy = p1 + y
                    cz = p2 + z
                    bk = bk_ref[pl.ds(kk * 16, 16)]
                    bx = st[pl.ds(kk * 16, 16)]
                    by = st[pl.ds((_NKP + kk) * 16, 16)]
                    bz = st[pl.ds((2 * _NKP + kk) * 16, 16)]
                    sk, sx = plsc.sort_key_val(key, cx)
                    _, sy = plsc.sort_key_val(key, cy)
                    _, sz = plsc.sort_key_val(key, cz)
                    rk = lax.rev(sk, (0,))
                    take = bk <= rk
                    lo_k = jnp.where(take, bk, rk)
                    lo_x = jnp.where(take, bx, lax.rev(sx, (0,)))
                    lo_y = jnp.where(take, by, lax.rev(sy, (0,)))
                    lo_z = jnp.where(take, bz, lax.rev(sz, (0,)))
                    nk, nx = plsc.sort_key_val(lo_k, lo_x)
                    _, ny = plsc.sort_key_val(lo_k, lo_y)
                    _, nz = plsc.sort_key_val(lo_k, lo_z)
                    bk_ref[pl.ds(kk * 16, 16)] = nk
                    st[pl.ds(kk * 16, 16)] = nx
                    st[pl.ds((_NKP + kk) * 16, 16)] = ny
                    st[pl.ds((2 * _NKP + kk) * 16, 16)] = nz
            return 0

        return lax.fori_loop(0, _C // 16, group_body, 0)

    lax.fori_loop(0, n_points // _C, chunk_body, 0)
    pltpu.sync_copy(st, out_hbm.at[b])


def _cluster_kernel(g_ref, o_ref):
    g = g_ref[...]
    third = g.shape[0] // 3
    x = g[:third]
    y = g[third:2 * third]
    z = g[2 * third:]
    valid = lax.broadcasted_iota(jnp.int32, x.shape, 1) < _K
    inv_k = jnp.float32(1.0 / _K)
    eps = jnp.float32(1e-9)

    def stats(v):
        mean = jnp.sum(jnp.where(valid, v, 0.0), axis=1, keepdims=True) * inv_k
        d = v - mean
        std = jnp.sqrt(
            jnp.sum(jnp.where(valid, d * d, 0.0), axis=1, keepdims=True) * inv_k)
        return d, std

    dx, sx = stats(x)
    dy, sy = stats(y)
    dz, sz = stats(z)
    inl = (valid & (jnp.abs(dx) <= sx + eps) & (jnp.abs(dy) <= sy + eps)
           & (jnp.abs(dz) <= sz + eps))
    w = inl.astype(jnp.float32)
    denom = jnp.sum(w, axis=1, keepdims=True) + jnp.float32(1e-8)
    ox = jnp.sum(jnp.where(valid, x, 0.0) * w, axis=1, keepdims=True) / denom
    oy = jnp.sum(jnp.where(valid, y, 0.0) * w, axis=1, keepdims=True) / denom
    oz = jnp.sum(jnp.where(valid, z, 0.0) * w, axis=1, keepdims=True) / denom
    o_ref[...] = jnp.concatenate([ox, oy, oz], axis=1)


def kernel(pcld_input, kpts_pre_input, cpt_pre_input, seg_pre_input):
    b, n = pcld_input.shape[0], pcld_input.shape[1]
    kpts_t = jnp.transpose(kpts_pre_input, (0, 3, 2, 1))
    cpt_t = jnp.transpose(cpt_pre_input, (0, 3, 2, 1)).reshape(b, 3, n)
    pcld_t = jnp.transpose(pcld_input, (2, 0, 1))
    seg_t = jnp.transpose(seg_pre_input, (0, 2, 1))

    mesh = plsc.VectorSubcoreMesh(core_axis_name="c", subcore_axis_name="s")
    scan = pl.kernel(
        functools.partial(_sc_scan_body, n),
        mesh=mesh,
        compiler_params=pltpu.CompilerParams(needs_layout_passes=False),
        out_type=jax.ShapeDtypeStruct((b, 3 * _NKP * 16), jnp.float32),
        scratch_types=[
            pltpu.VMEM((3, 8, _C), jnp.float32),
            pltpu.VMEM((3, _C), jnp.float32),
            pltpu.VMEM((3, _C), jnp.float32),
            pltpu.VMEM((2, _C), jnp.float32),
            pltpu.VMEM((3 * _NKP * 16,), jnp.float32),
            pltpu.VMEM((_NKP * 16,), jnp.float32),
            pltpu.SemaphoreType.DMA,
        ],
    )
    g = scan(kpts_t, cpt_t, pcld_t, seg_t)
    g2 = g.reshape(b, 3, _NKP, 16).transpose(1, 0, 2, 3).reshape(3 * b * _NKP, 16)

    out = pl.pallas_call(
        _cluster_kernel,
        out_shape=jax.ShapeDtypeStruct((b * _NKP, 3), jnp.float32),
    )(g2)
    return out.reshape(b, _NKP, 3)


# speedup vs baseline: 3.6082x; 1.0352x over previous
import functools

import jax
import jax.numpy as jnp
from jax import lax
from jax.experimental import pallas as pl
from jax.experimental.pallas import tpu as pltpu
from jax.experimental.pallas import tpu_sc as plsc

_K = 10
_NKP = 9
_C = 2048
_BIG = 1e30


def _sc_scan_body(n_points, kpts_hbm, cpt_hbm, pcld_hbm, seg_hbm, out_hbm,
                  kc, cc, pc, sgc, st, bk_ref, sem):
    b = lax.axis_index("s") * 2 + lax.axis_index("c")
    it = lax.iota(jnp.int32, 16)
    big = jnp.float32(_BIG)

    for kk in range(_NKP):
        bk_ref[pl.ds(kk * 16, 16)] = jnp.full((16,), big, jnp.float32)

    def chunk_body(q, carry):
        s = q * _C
        cps = (pltpu.async_copy(kpts_hbm.at[b, :, :, pl.ds(s, _C)], kc, sem),
               pltpu.async_copy(cpt_hbm.at[b, :, pl.ds(s, _C)], cc, sem),
               pltpu.async_copy(pcld_hbm.at[:, b, pl.ds(s, _C)], pc, sem),
               pltpu.async_copy(seg_hbm.at[b, :, pl.ds(s, _C)], sgc, sem))
        for cp in cps:
            cp.wait()

        def group_body(j, carry):
            sl = pl.ds(j * 16, 16)
            gp = q * _C + j * 16 + it
            s0 = sgc[0, sl]
            s1 = sgc[1, sl]
            obj = s1 > s0
            sent = jnp.float32(1e18) + gp.astype(jnp.float32) * jnp.float32(1e12)
            keys = []
            hit = None
            for kk in range(_NKP):
                if kk < 8:
                    x = kc[0, kk, sl]
                    y = kc[1, kk, sl]
                    z = kc[2, kk, sl]
                else:
                    x = cc[0, sl]
                    y = cc[1, sl]
                    z = cc[2, sl]
                n2 = x * x + y * y + z * z
                key = jnp.where(obj, n2, sent)
                keys.append((key, x, y, z))
                thv = plsc.load_gather(
                    bk_ref, [jnp.full((16,), kk * 16 + 9, jnp.int32)])
                h = key < thv
                hit = h if hit is None else (hit | h)

            @pl.when(jnp.any(hit))
            def _(keys=keys, sl=sl):
                p0 = pc[0, sl]
                p1 = pc[1, sl]
                p2 = pc[2, sl]
                for kk in range(_NKP):
                    key, x, y, z = keys[kk]
                    cx = p0 + x
                    cy = p1 + y
                    cz = p2 + z
                    bk = bk_ref[pl.ds(kk * 16, 16)]
                    bx = st[pl.ds(kk * 16, 16)]
                    by = st[pl.ds((_NKP + kk) * 16, 16)]
                    bz = st[pl.ds((2 * _NKP + kk) * 16, 16)]
                    sk, sx = plsc.sort_key_val(key, cx)
                    _, sy = plsc.sort_key_val(key, cy)
                    _, sz = plsc.sort_key_val(key, cz)
                    rk = lax.rev(sk, (0,))
                    take = bk <= rk
                    lo_k = jnp.where(take, bk, rk)
                    lo_x = jnp.where(take, bx, lax.rev(sx, (0,)))
                    lo_y = jnp.where(take, by, lax.rev(sy, (0,)))
                    lo_z = jnp.where(take, bz, lax.rev(sz, (0,)))
                    nk, nx = plsc.sort_key_val(lo_k, lo_x)
                    _, ny = plsc.sort_key_val(lo_k, lo_y)
                    _, nz = plsc.sort_key_val(lo_k, lo_z)
                    bk_ref[pl.ds(kk * 16, 16)] = nk
                    st[pl.ds(kk * 16, 16)] = nx
                    st[pl.ds((_NKP + kk) * 16, 16)] = ny
                    st[pl.ds((2 * _NKP + kk) * 16, 16)] = nz
            return 0

        return lax.fori_loop(0, _C // 16, group_body, 0)

    lax.fori_loop(0, n_points // _C, chunk_body, 0)
    pltpu.sync_copy(st, out_hbm.at[b])


def _cluster_kernel(g_ref, o_ref):
    g = g_ref[...]
    third = g.shape[0] // 3
    x = g[:third]
    y = g[third:2 * third]
    z = g[2 * third:]
    valid = lax.broadcasted_iota(jnp.int32, x.shape, 1) < _K
    inv_k = jnp.float32(1.0 / _K)
    eps = jnp.float32(1e-9)

    def stats(v):
        mean = jnp.sum(jnp.where(valid, v, 0.0), axis=1, keepdims=True) * inv_k
        d = v - mean
        std = jnp.sqrt(
            jnp.sum(jnp.where(valid, d * d, 0.0), axis=1, keepdims=True) * inv_k)
        return d, std

    dx, sx = stats(x)
    dy, sy = stats(y)
    dz, sz = stats(z)
    inl = (valid & (jnp.abs(dx) <= sx + eps) & (jnp.abs(dy) <= sy + eps)
           & (jnp.abs(dz) <= sz + eps))
    w = inl.astype(jnp.float32)
    denom = jnp.sum(w, axis=1, keepdims=True) + jnp.float32(1e-8)
    ox = jnp.sum(jnp.where(valid, x, 0.0) * w, axis=1, keepdims=True) / denom
    oy = jnp.sum(jnp.where(valid, y, 0.0) * w, axis=1, keepdims=True) / denom
    oz = jnp.sum(jnp.where(valid, z, 0.0) * w, axis=1, keepdims=True) / denom
    o_ref[...] = jnp.concatenate([ox, oy, oz], axis=1)


def kernel(pcld_input, kpts_pre_input, cpt_pre_input, seg_pre_input):
    b, n = pcld_input.shape[0], pcld_input.shape[1]
    kpts_t = jnp.transpose(kpts_pre_input, (0, 3, 2, 1))
    cpt_t = jnp.transpose(cpt_pre_input, (0, 3, 2, 1)).reshape(b, 3, n)
    pcld_t = jnp.transpose(pcld_input, (2, 0, 1))
    seg_t = jnp.transpose(seg_pre_input, (0, 2, 1))

    mesh = plsc.VectorSubcoreMesh(core_axis_name="c", subcore_axis_name="s")
    scan = pl.kernel(
        functools.partial(_sc_scan_body, n),
        mesh=mesh,
        compiler_params=pltpu.CompilerParams(needs_layout_passes=False),
        out_type=jax.ShapeDtypeStruct((b, 3 * _NKP * 16), jnp.float32),
        scratch_types=[
            pltpu.VMEM((3, 8, _C), jnp.float32),
            pltpu.VMEM((3, _C), jnp.float32),
            pltpu.VMEM((3, _C), jnp.float32),
            pltpu.VMEM((2, _C), jnp.float32),
            pltpu.VMEM((3 * _NKP * 16,), jnp.float32),
            pltpu.VMEM((_NKP * 16,), jnp.float32),
            pltpu.SemaphoreType.DMA,
        ],
    )
    g = scan(kpts_t, cpt_t, pcld_t, seg_t)
    g2 = g.reshape(b, 3, _NKP, 16).transpose(1, 0, 2, 3).reshape(3 * b * _NKP, 16)

    out = pl.pallas_call(
        _cluster_kernel,
        out_shape=jax.ShapeDtypeStruct((b * _NKP, 3), jnp.float32),
    )(g2)
    return out.reshape(b, _NKP, 3)
